# baseline (speedup 1.0000x reference)
import functools

import jax
import jax.numpy as jnp
from jax import lax
from jax.experimental import pallas as pl
from jax.experimental.pallas import tpu as pltpu


def _pick_tile(v):
    for tv in (1024, 512, 256, 128, 64, 32, 16, 8):
        if v % tv == 0:
            return tv
    return v


def _hybrid_kernel(sids_ref, order_ref, starts_ref,
                   w0_ref, w1_ref, w2_ref, w0a, w1a, w2a,
                   m_in_ref, m_out_ref, mtok_ref, s0_ref,
                   out_hbm, merged, gbuf, gout, sems, gsems, gwsems,
                   *, tv, nv, kv, g, t):
    i = pl.program_id(0)
    slot = lax.rem(i, 2)
    s0 = s0_ref[0]
    s_kv = starts_ref[kv]
    gstart = (s_kv // g) * g

    def wait_rows(sem, n):
        def body8(_, c):
            pltpu.make_async_copy(
                merged.at[0, pl.ds(0, 8)], out_hbm.at[pl.ds(0, 8)], sem).wait()
            return c
        lax.fori_loop(0, n >> 3, body8, 0)

        def body1(_, c):
            pltpu.make_async_copy(
                merged.at[0, pl.ds(0, 1)], out_hbm.at[pl.ds(0, 1)], sem).wait()
            return c
        lax.fori_loop(0, n & 7, body1, 0)

    @pl.when(i >= 2)
    def _():
        wait_rows(sems.at[slot], starts_ref[i - 1] - starts_ref[i - 2])

    merged[slot] = (w0_ref[...] * s0 + w1_ref[...] * m_in_ref[...]
                    + w2_ref[...] * m_out_ref[...])

    lo = starts_ref[i]
    base = i * tv
    n = starts_ref[i + 1] - lo

    def scat_row(idx):
        pltpu.make_async_copy(
            merged.at[slot, pl.ds(sids_ref[idx] - base, 1)],
            out_hbm.at[pl.ds(order_ref[idx], 1)], sems.at[slot]).start()

    def scat4(c, carry):
        for u in range(4):
            scat_row(lo + c * 4 + u)
        return carry
    lax.fori_loop(0, n >> 2, scat4, 0)

    def scat1(k, carry):
        scat_row(lo + (n & ~3) + k)
        return carry
    lax.fori_loop(0, n & 3, scat1, 0)

    def cnt(j):
        return jnp.clip(t - (gstart + j * g), 0, g)

    def issue_reads(j, sl):
        cb = gstart + j * g

        def rd(idx, k):
            row = sids_ref[idx]
            pltpu.make_async_copy(
                w0a.at[pl.ds(row, 1)], gbuf.at[sl, pl.ds(k, 1)],
                gsems.at[sl]).start()
            pltpu.make_async_copy(
                w1a.at[pl.ds(row, 1)], gbuf.at[sl, pl.ds(g + k, 1)],
                gsems.at[sl]).start()
            pltpu.make_async_copy(
                w2a.at[pl.ds(row, 1)], gbuf.at[sl, pl.ds(2 * g + k, 1)],
                gsems.at[sl]).start()

        m = cnt(j)

        def rd4(c, carry):
            for u in range(4):
                rd(cb + c * 4 + u, c * 4 + u)
            return carry
        lax.fori_loop(0, m >> 2, rd4, 0)

        def rd1(k, carry):
            rd(cb + (m & ~3) + k, (m & ~3) + k)
            return carry
        lax.fori_loop(0, m & 3, rd1, 0)

    @pl.when(i == 0)
    def _():
        issue_reads(0, 0)

    issue_reads(i + 1, 1 - slot)

    m = cnt(i)
    wait_rows(gsems.at[slot], 3 * m)

    @pl.when(i >= 2)
    def _():
        wait_rows(gwsems.at[slot], cnt(i - 2))

    @pl.when(m > 0)
    def _():
        gout[slot] = (gbuf[slot, 0:g] * s0 + gbuf[slot, g:2 * g]
                      * m_in_ref[...] + gbuf[slot, 2 * g:3 * g]
                      * mtok_ref[...])

    def gw(k, carry):
        pltpu.make_async_copy(
            gout.at[slot, pl.ds(k, 1)],
            out_hbm.at[pl.ds(order_ref[gstart + i * g + k], 1)],
            gwsems.at[slot]).start()
        return carry
    lax.fori_loop(0, m, gw, 0)

    @pl.when(i == kv - 1)
    def _():
        if kv >= 2:
            wait_rows(sems.at[1 - slot],
                      starts_ref[kv - 1] - starts_ref[kv - 2])
            wait_rows(gwsems.at[1 - slot], cnt(kv - 2))
        wait_rows(sems.at[slot], starts_ref[kv] - starts_ref[kv - 1])
        wait_rows(gwsems.at[slot], cnt(kv - 1))
        wait_rows(gsems.at[1 - slot], 3 * cnt(kv))


def kernel(input_ids, w0, w1, w2, scalar_mask, vec_in_mask, vec_out_mask):
    B, S = input_ids.shape
    V, H = w0.shape
    dtype = w0.dtype
    T = B * S

    ids = input_ids.reshape(T).astype(jnp.int32)
    sids, order = lax.sort([ids, lax.iota(jnp.int32, T)], num_keys=1)
    tv = _pick_tile(V)
    nv = V // tv
    kv = nv if nv < 8 else max(2, (nv * 3) // 5)
    g = max(8, (-(-T // kv) + 7) // 8 * 8) if kv < nv else 8

    bounds = jnp.arange(nv + 1, dtype=jnp.int32) * tv
    starts = jnp.sum(sids[None, :] < bounds[:, None], axis=1,
                     dtype=jnp.int32)

    m_in = jnp.asarray(vec_in_mask, dtype).reshape(1, H)
    m_out = jnp.asarray(vec_out_mask, dtype).reshape(V, 1)
    s0 = jnp.asarray(scalar_mask, dtype).reshape(1)
    lp = g * (-(-T // g) + kv + 1)
    mtok = jnp.pad(jnp.take(m_out.reshape(V), sids), (0, lp - T)).reshape(
        lp, 1)

    def tile_idx(i, *_):
        return (i, 0)

    grid_spec = pltpu.PrefetchScalarGridSpec(
        num_scalar_prefetch=3,
        grid=(kv,),
        in_specs=[
            pl.BlockSpec((tv, H), tile_idx),
            pl.BlockSpec((tv, H), tile_idx),
            pl.BlockSpec((tv, H), tile_idx),
            pl.BlockSpec(memory_space=pl.ANY),
            pl.BlockSpec(memory_space=pl.ANY),
            pl.BlockSpec(memory_space=pl.ANY),
            pl.BlockSpec((1, H), lambda i, *_: (0, 0)),
            pl.BlockSpec((tv, 1), tile_idx),
            pl.BlockSpec((g, 1),
                         lambda i, sids, order, starts: (starts[kv] // g + i,
                                                         0)),
            pl.BlockSpec(memory_space=pltpu.MemorySpace.SMEM),
        ],
        out_specs=pl.BlockSpec(memory_space=pl.ANY),
        scratch_shapes=[
            pltpu.VMEM((2, tv, H), dtype),
            pltpu.VMEM((2, 3 * g, H), dtype),
            pltpu.VMEM((2, g, H), dtype),
            pltpu.SemaphoreType.DMA((2,)),
            pltpu.SemaphoreType.DMA((2,)),
            pltpu.SemaphoreType.DMA((2,)),
        ],
    )
    out = pl.pallas_call(
        functools.partial(_hybrid_kernel, tv=tv, nv=nv, kv=kv, g=g, t=T),
        out_shape=jax.ShapeDtypeStruct((T, H), dtype),
        grid_spec=grid_spec,
        compiler_params=pltpu.CompilerParams(
            dimension_semantics=("arbitrary",),
            vmem_limit_bytes=56 * 1024 * 1024,
            disable_bounds_checks=True),
        name="hybrid_stream_gather",
    )(sids, order, starts, w0, w1, w2, w0, w1, w2, m_in, m_out, mtok, s0)
    return out.reshape(B, S, H)


# speedup vs baseline: 1.3184x; 1.3184x over previous
import functools

import jax
import jax.numpy as jnp
from jax import lax
from jax.experimental import pallas as pl
from jax.experimental.pallas import tpu as pltpu


def _pick_tile(v):
    for tv in (1024, 512, 256, 128, 64, 32, 16, 8):
        if v % tv == 0:
            return tv
    return v


def _merge_scatter_kernel(sids_ref, order_ref, starts_ref,
                          w0_ref, w1_ref, w2_ref, m_in_ref, m_out_ref, s0_ref,
                          out_hbm, merged, sems, *, tv, nv, seq):
    i = pl.program_id(0)
    slot = lax.rem(i, 2)

    def wait_rows(n, sl):
        def body8(_, c):
            pltpu.make_async_copy(
                merged.at[sl, pl.ds(0, 8)], merged.at[sl, pl.ds(0, 8)],
                sems.at[sl]).wait()
            return c
        lax.fori_loop(0, n >> 3, body8, 0)

        def body1(_, c):
            pltpu.make_async_copy(
                merged.at[sl, pl.ds(0, 1)], merged.at[sl, pl.ds(0, 1)],
                sems.at[sl]).wait()
            return c
        lax.fori_loop(0, n & 7, body1, 0)

    @pl.when(i >= 2)
    def _():
        wait_rows(starts_ref[i - 1] - starts_ref[i - 2], slot)

    merged[slot] = (w0_ref[...] * s0_ref[0] + w1_ref[...] * m_in_ref[...]
                    + w2_ref[...] * m_out_ref[...])

    lo = starts_ref[i]
    base = i * tv

    n = starts_ref[i + 1] - lo

    def start_row(idx):
        row = sids_ref[idx] - base
        tok = order_ref[idx]
        pltpu.make_async_copy(
            merged.at[slot, pl.ds(row, 1)],
            out_hbm.at[tok // seq, pl.ds(lax.rem(tok, seq), 1)],
            sems.at[slot]).start()

    def issue4(c, carry):
        for u in range(4):
            start_row(lo + c * 4 + u)
        return carry
    lax.fori_loop(0, n >> 2, issue4, 0)

    def issue1(k, carry):
        start_row(lo + (n & ~3) + k)
        return carry
    lax.fori_loop(0, n & 3, issue1, 0)

    if nv >= 2:
        @pl.when(i == nv - 1)
        def _():
            wait_rows(starts_ref[nv - 1] - starts_ref[nv - 2], 1 - slot)
            wait_rows(starts_ref[nv] - starts_ref[nv - 1], slot)
    else:
        wait_rows(starts_ref[1] - starts_ref[0], slot)


def kernel(input_ids, w0, w1, w2, scalar_mask, vec_in_mask, vec_out_mask):
    B, S = input_ids.shape
    V, H = w0.shape
    dtype = w0.dtype
    T = B * S

    ids = input_ids.reshape(T).astype(jnp.int32)
    sids, order = lax.sort([ids, lax.iota(jnp.int32, T)], num_keys=1)
    tv = _pick_tile(V)
    nv = V // tv
    bounds = jnp.arange(nv + 1, dtype=jnp.int32) * tv
    starts = jnp.sum(sids[None, :] < bounds[:, None], axis=1,
                     dtype=jnp.int32)

    m_in = jnp.asarray(vec_in_mask, dtype).reshape(1, H)
    m_out = jnp.asarray(vec_out_mask, dtype).reshape(V, 1)
    s0 = jnp.asarray(scalar_mask, dtype).reshape(1)

    grid_spec = pltpu.PrefetchScalarGridSpec(
        num_scalar_prefetch=3,
        grid=(nv,),
        in_specs=[
            pl.BlockSpec((tv, H), lambda i, *_: (i, 0)),
            pl.BlockSpec((tv, H), lambda i, *_: (i, 0)),
            pl.BlockSpec((tv, H), lambda i, *_: (i, 0)),
            pl.BlockSpec((1, H), lambda i, *_: (0, 0)),
            pl.BlockSpec((tv, 1), lambda i, *_: (i, 0)),
            pl.BlockSpec(memory_space=pltpu.MemorySpace.SMEM),
        ],
        out_specs=pl.BlockSpec(memory_space=pl.ANY),
        scratch_shapes=[
            pltpu.VMEM((2, tv, H), dtype),
            pltpu.SemaphoreType.DMA((2,)),
        ],
    )
    return pl.pallas_call(
        functools.partial(_merge_scatter_kernel, tv=tv, nv=nv, seq=S),
        out_shape=jax.ShapeDtypeStruct((B, S, H), dtype),
        grid_spec=grid_spec,
        compiler_params=pltpu.CompilerParams(
            dimension_semantics=("arbitrary",),
            disable_bounds_checks=True),
        name="merge_scatter_stream",
    )(sids, order, starts, w0, w1, w2, m_in, m_out, s0)


# speedup vs baseline: 1.4496x; 1.0995x over previous
import functools

import jax
import jax.numpy as jnp
from jax import lax
from jax.experimental import pallas as pl
from jax.experimental.pallas import tpu as pltpu


def _pick_tile(v):
    for tv in (1024, 512, 256, 128, 64, 32, 16, 8):
        if v % tv == 0:
            return tv
    return v


def _merge_scatter_kernel(sids_ref, order_ref, starts_ref,
                          w0_ref, w1_ref, w2_ref, m_in_ref, m_out_ref, s0_ref,
                          out_hbm, merged, sems, *, tv, nv):
    i = pl.program_id(0)
    slot = lax.rem(i, 2)

    def wait_rows(n, sl):
        def body8(_, c):
            pltpu.make_async_copy(
                merged.at[sl, pl.ds(0, 8)], out_hbm.at[pl.ds(0, 8)],
                sems.at[sl]).wait()
            return c
        lax.fori_loop(0, n >> 3, body8, 0)

        def body1(_, c):
            pltpu.make_async_copy(
                merged.at[sl, pl.ds(0, 1)], out_hbm.at[pl.ds(0, 1)],
                sems.at[sl]).wait()
            return c
        lax.fori_loop(0, n & 7, body1, 0)

    @pl.when(i >= 2)
    def _():
        wait_rows(starts_ref[i - 1] - starts_ref[i - 2], slot)

    merged[slot] = (w0_ref[...] * s0_ref[0] + w1_ref[...] * m_in_ref[...]
                    + w2_ref[...] * m_out_ref[...])

    lo = starts_ref[i]
    base = i * tv

    n = starts_ref[i + 1] - lo

    def start_row(idx):
        row = sids_ref[idx] - base
        tok = order_ref[idx]
        pltpu.make_async_copy(
            merged.at[slot, pl.ds(row, 1)], out_hbm.at[pl.ds(tok, 1)],
            sems.at[slot]).start()

    def issue4(c, carry):
        for u in range(4):
            start_row(lo + c * 4 + u)
        return carry
    lax.fori_loop(0, n >> 2, issue4, 0)

    def issue1(k, carry):
        start_row(lo + (n & ~3) + k)
        return carry
    lax.fori_loop(0, n & 3, issue1, 0)

    if nv >= 2:
        @pl.when(i == nv - 1)
        def _():
            wait_rows(starts_ref[nv - 1] - starts_ref[nv - 2], 1 - slot)
            wait_rows(starts_ref[nv] - starts_ref[nv - 1], slot)
    else:
        wait_rows(starts_ref[1] - starts_ref[0], slot)


def kernel(input_ids, w0, w1, w2, scalar_mask, vec_in_mask, vec_out_mask):
    B, S = input_ids.shape
    V, H = w0.shape
    dtype = w0.dtype
    T = B * S

    ids = input_ids.reshape(T).astype(jnp.int32)
    sids, order = lax.sort([ids, lax.iota(jnp.int32, T)], num_keys=1)
    tv = _pick_tile(V)
    nv = V // tv
    bounds = jnp.arange(nv + 1, dtype=jnp.int32) * tv
    starts = jnp.sum(sids[None, :] < bounds[:, None], axis=1,
                     dtype=jnp.int32)

    m_in = jnp.asarray(vec_in_mask, dtype).reshape(1, H)
    m_out = jnp.asarray(vec_out_mask, dtype).reshape(V, 1)
    s0 = jnp.asarray(scalar_mask, dtype).reshape(1)

    grid_spec = pltpu.PrefetchScalarGridSpec(
        num_scalar_prefetch=3,
        grid=(nv,),
        in_specs=[
            pl.BlockSpec((tv, H), lambda i, *_: (i, 0)),
            pl.BlockSpec((tv, H), lambda i, *_: (i, 0)),
            pl.BlockSpec((tv, H), lambda i, *_: (i, 0)),
            pl.BlockSpec((1, H), lambda i, *_: (0, 0)),
            pl.BlockSpec((tv, 1), lambda i, *_: (i, 0)),
            pl.BlockSpec(memory_space=pltpu.MemorySpace.SMEM),
        ],
        out_specs=pl.BlockSpec(memory_space=pl.ANY),
        scratch_shapes=[
            pltpu.VMEM((2, tv, H), dtype),
            pltpu.SemaphoreType.DMA((2,)),
        ],
    )
    out = pl.pallas_call(
        functools.partial(_merge_scatter_kernel, tv=tv, nv=nv),
        out_shape=jax.ShapeDtypeStruct((T, H), dtype),
        grid_spec=grid_spec,
        compiler_params=pltpu.CompilerParams(
            dimension_semantics=("arbitrary",),
            disable_bounds_checks=True),
        name="merge_scatter_stream",
    )(sids, order, starts, w0, w1, w2, m_in, m_out, s0)
    return out.reshape(B, S, H)


# speedup vs baseline: 1.5322x; 1.0570x over previous
import functools

import jax
import jax.numpy as jnp
from jax import lax
from jax.experimental import pallas as pl
from jax.experimental.pallas import tpu as pltpu


def _pick_tile(v):
    for tv in (1024, 512, 256, 128, 64, 32, 16, 8):
        if v % tv == 0:
            return tv
    return v


def _merge_scatter_kernel(sids_ref, order_ref, starts_ref,
                          w0_ref, w1_ref, w2_ref, m_in_ref, m_out_ref, s0_ref,
                          out_hbm, merged, sems, *, tv, nv):
    i = pl.program_id(0)
    slot = lax.rem(i, 2)

    def wait_rows(n, sl):
        def body8(_, c):
            pltpu.make_async_copy(
                merged.at[sl, pl.ds(0, 8)], out_hbm.at[pl.ds(0, 8)],
                sems.at[sl]).wait()
            return c
        lax.fori_loop(0, n >> 3, body8, 0)

        def body1(_, c):
            pltpu.make_async_copy(
                merged.at[sl, pl.ds(0, 1)], out_hbm.at[pl.ds(0, 1)],
                sems.at[sl]).wait()
            return c
        lax.fori_loop(0, n & 7, body1, 0)

    @pl.when(i >= 2)
    def _():
        wait_rows(starts_ref[i - 1] - starts_ref[i - 2], slot)

    m_col = jnp.transpose(m_out_ref[0], (1, 0))
    merged[slot] = (w0_ref[...] * s0_ref[0] + w1_ref[...] * m_in_ref[...]
                    + w2_ref[...] * m_col)

    lo = starts_ref[i]
    base = i * tv

    n = starts_ref[i + 1] - lo

    def start_row(idx):
        row = sids_ref[idx] - base
        tok = order_ref[idx]
        pltpu.make_async_copy(
            merged.at[slot, pl.ds(row, 1)], out_hbm.at[pl.ds(tok, 1)],
            sems.at[slot]).start()

    def issue4(c, carry):
        for u in range(4):
            start_row(lo + c * 4 + u)
        return carry
    lax.fori_loop(0, n >> 2, issue4, 0)

    def issue1(k, carry):
        start_row(lo + (n & ~3) + k)
        return carry
    lax.fori_loop(0, n & 3, issue1, 0)

    if nv >= 2:
        @pl.when(i == nv - 1)
        def _():
            wait_rows(starts_ref[nv - 1] - starts_ref[nv - 2], 1 - slot)
            wait_rows(starts_ref[nv] - starts_ref[nv - 1], slot)
    else:
        wait_rows(starts_ref[1] - starts_ref[0], slot)


def kernel(input_ids, w0, w1, w2, scalar_mask, vec_in_mask, vec_out_mask):
    B, S = input_ids.shape
    V, H = w0.shape
    dtype = w0.dtype
    T = B * S

    ids = input_ids.reshape(T).astype(jnp.int32)
    sids, order = lax.sort([ids, lax.iota(jnp.int32, T)], num_keys=1)
    tv = _pick_tile(V)
    nv = V // tv
    bounds = jnp.arange(nv + 1, dtype=jnp.int32) * tv
    starts = jnp.sum(sids[None, :] < bounds[:, None], axis=1,
                     dtype=jnp.int32)

    m_in = jnp.asarray(vec_in_mask, dtype).reshape(1, H)
    m_out = jnp.asarray(vec_out_mask, dtype).reshape(nv, 1, tv)
    s0 = jnp.asarray(scalar_mask, dtype).reshape(1)

    grid_spec = pltpu.PrefetchScalarGridSpec(
        num_scalar_prefetch=3,
        grid=(nv,),
        in_specs=[
            pl.BlockSpec((tv, H), lambda i, *_: (i, 0)),
            pl.BlockSpec((tv, H), lambda i, *_: (i, 0)),
            pl.BlockSpec((tv, H), lambda i, *_: (i, 0)),
            pl.BlockSpec((1, H), lambda i, *_: (0, 0)),
            pl.BlockSpec((1, 1, tv), lambda i, *_: (i, 0, 0)),
            pl.BlockSpec(memory_space=pltpu.MemorySpace.SMEM),
        ],
        out_specs=pl.BlockSpec(memory_space=pl.ANY),
        scratch_shapes=[
            pltpu.VMEM((2, tv, H), dtype),
            pltpu.SemaphoreType.DMA((2,)),
        ],
    )
    out = pl.pallas_call(
        functools.partial(_merge_scatter_kernel, tv=tv, nv=nv),
        out_shape=jax.ShapeDtypeStruct((T, H), dtype),
        grid_spec=grid_spec,
        compiler_params=pltpu.CompilerParams(
            dimension_semantics=("arbitrary",),
            disable_bounds_checks=True),
        name="merge_scatter_stream",
    )(sids, order, starts, w0, w1, w2, m_in, m_out, s0)
    return out.reshape(B, S, H)
